# baseline (speedup 1.0000x reference)
import math

import jax
import jax.numpy as jnp
from jax.experimental import pallas as pl

NCLASSES = 3
NMS_THR = 0.25
SCORE_THR = 0.1
MAX_NUM = 500


def _identity_kernel(x_ref, o_ref):
    o_ref[...] = x_ref[...]


def _nms_keep(boxes5, scores):
    N = boxes5.shape[0]
    x1, y1, x2, y2 = boxes5[:, 0], boxes5[:, 1], boxes5[:, 2], boxes5[:, 3]
    areas = (x2 - x1) * (y2 - y1)
    idxs = jnp.arange(N)

    def cond(state):
        alive, _ = state
        return jnp.any(alive)

    def body(state):
        alive, kept = state
        s_eff = jnp.where(alive, scores, -jnp.inf)
        i = jnp.argmax(s_eff)
        xx1 = jnp.maximum(x1[i], x1)
        yy1 = jnp.maximum(y1[i], y1)
        xx2 = jnp.minimum(x2[i], x2)
        yy2 = jnp.minimum(y2[i], y2)
        inter = jnp.maximum(0.0, xx2 - xx1) * jnp.maximum(0.0, yy2 - yy1)
        iou = inter / (areas[i] + areas - inter + 1e-8)
        kept = kept | (idxs == i)
        alive = alive & (iou <= NMS_THR) & (idxs != i)
        return alive, kept

    alive0 = scores > SCORE_THR
    kept0 = jnp.zeros((N,), dtype=bool)
    _, kept = jax.lax.while_loop(cond, body, (alive0, kept0))
    return kept


def _process_one(r):
    N = r.shape[0]
    bbox_pred = r[:, :7]
    bbox_cls_pred = r[:, 7:10]
    bbox_dir_cls_pred = r[:, 10]
    xy = bbox_pred[:, 0:2]
    lw = bbox_pred[:, 3:5]
    bbox_pred2d = jnp.concatenate([xy - lw / 2, xy + lw / 2, bbox_pred[:, 6:7]], axis=1)

    kept = jax.vmap(lambda s: _nms_keep(bbox_pred2d, s))(bbox_cls_pred.T)

    theta = bbox_pred[:, 6]
    theta = theta - jnp.floor(theta / math.pi + 1) * math.pi
    theta = theta + (1.0 - bbox_dir_cls_pred) * math.pi
    tb = jnp.concatenate([bbox_pred[:, :6], theta[:, None]], axis=1)

    kept_flat = kept.reshape(-1)
    scores_flat = bbox_cls_pred.T.reshape(-1)
    cls_flat = jnp.repeat(jnp.arange(NCLASSES), N)
    flat_idx = jnp.arange(NCLASSES * N)
    neg_s = -scores_flat

    cls_eff = jnp.where(kept_flat, cls_flat, NCLASSES)
    perm1 = jnp.lexsort((flat_idx, neg_s, cls_eff))
    pos = jnp.zeros_like(flat_idx).at[perm1].set(flat_idx)
    neg_s_eff = jnp.where(kept_flat, neg_s, jnp.inf)
    perm2 = jnp.lexsort((pos, neg_s_eff))

    total = jnp.sum(kept_flat)
    sel = jnp.where(total > MAX_NUM, perm2[:MAX_NUM], perm1[:MAX_NUM])
    j = sel % N
    c = sel // N
    valid_slot = jnp.arange(MAX_NUM) < total

    rows = jnp.concatenate([tb[j], scores_flat[sel][:, None]], axis=1)
    out8 = jnp.where(valid_slot[:, None], rows, jnp.float32(0.0)).astype(jnp.float32)
    lab = jnp.where(valid_slot, c.astype(jnp.int32), jnp.int32(-1))
    return out8, lab


def kernel(results):
    flat = results.reshape(6875, 128)
    flat = pl.pallas_call(
        _identity_kernel,
        out_shape=jax.ShapeDtypeStruct(flat.shape, flat.dtype),
    )(flat)
    results = flat.reshape(results.shape)
    outs, labels_out = jax.vmap(_process_one)(results)
    return (outs, labels_out)


# speedup vs baseline: 14.7825x; 14.7825x over previous
import math

import jax
import jax.numpy as jnp
from jax.experimental import pallas as pl
from jax.experimental.pallas import tpu as pltpu

NCLASSES = 3
NMS_THR = 0.25
SCORE_THR = 0.1
MAX_NUM = 500
OUTPAD = 512
NEG_INF = float("-inf")


def _make_kernel(NP):
    R = NP // 128

    def body(cols_ref, rows_ref, tb_ref, out_ref, lab_ref,
             sa_ref, k_ref, skey_ref, p_ref, s2_ref):
        jiota = jax.lax.broadcasted_iota(jnp.int32, (R, 128), 0) * 128 + \
            jax.lax.broadcasted_iota(jnp.int32, (R, 128), 1)

        X1 = cols_ref[0, 0]
        Y1 = cols_ref[0, 1]
        X2 = cols_ref[0, 2]
        Y2 = cols_ref[0, 3]
        AR = cols_ref[0, 4]
        pos_ext = ((X2 - X1) > 0.0) & ((Y2 - Y1) > 0.0)

        total = jnp.float32(0.0)
        for c in range(NCLASSES):
            S = cols_ref[0, 5 + c]
            thrm = S > SCORE_THR
            cand = thrm & pos_ext
            sa_ref[...] = jnp.where(cand, S, NEG_INF)
            k_ref[...] = jnp.zeros((R, 128), jnp.float32)

            def wbody(m):
                sa = sa_ref[...]
                mask = sa == m
                ii = jnp.min(jnp.where(mask, jiota, NP)).astype(jnp.int32)
                row = rows_ref[0, pl.ds(ii, 1), :]
                x1s = row[0:1, 0:1]
                y1s = row[0:1, 1:2]
                x2s = row[0:1, 2:3]
                y2s = row[0:1, 3:4]
                ars = row[0:1, 4:5]
                xx1 = jnp.maximum(X1, x1s)
                yy1 = jnp.maximum(Y1, y1s)
                xx2 = jnp.minimum(X2, x2s)
                yy2 = jnp.minimum(Y2, y2s)
                inter = jnp.maximum(0.0, xx2 - xx1) * jnp.maximum(0.0, yy2 - yy1)
                iou = inter / (ars + AR - inter + 1e-8)
                sup = (iou > NMS_THR) | (jiota == ii)
                sa_new = jnp.where(sup, NEG_INF, sa)
                sa_ref[...] = sa_new
                k_ref[...] = jnp.where(jiota == ii, 1.0, k_ref[...])
                return jnp.max(sa_new)

            m0 = jnp.max(sa_ref[...])
            jax.lax.while_loop(lambda m: m > NEG_INF, wbody, m0)

            kept = (k_ref[...] > 0.0) | (thrm & jnp.logical_not(pos_ext))
            skey_ref[c] = jnp.where(kept, S, NEG_INF)
            total = total + jnp.sum(kept.astype(jnp.float32))

        br = total > float(MAX_NUM)
        for c in range(NCLASSES):
            sk = skey_ref[c]
            iskept = sk > NEG_INF
            p_ref[c] = jnp.where(br, sk,
                                 jnp.where(iskept, jnp.float32(-c), NEG_INF))
            s2_ref[c] = jnp.where(br, jnp.full((R, 128), jnp.float32(-c)), sk)

        out_ref[...] = jnp.zeros((1, OUTPAD, 8), jnp.float32)
        lab_ref[...] = jnp.full((1, OUTPAD, 8), -1.0, jnp.float32)

        c3iota = jax.lax.broadcasted_iota(jnp.int32, (NCLASSES, R, 128), 0)
        f3iota = c3iota * NP + jiota[None]
        lane8 = jax.lax.broadcasted_iota(jnp.int32, (1, 8), 1)

        def selbody(k, carry):
            pall = p_ref[...]
            m1 = jnp.max(pall)
            valid = m1 > NEG_INF
            mask1 = pall == m1
            sall = s2_ref[...]
            m2 = jnp.max(jnp.where(mask1, sall, NEG_INF))
            mask2 = mask1 & (sall == m2)
            ii = jnp.min(jnp.where(mask2, f3iota, NCLASSES * NP))
            ii = jnp.where(valid, ii, 0).astype(jnp.int32)
            c_sel = ii // NP
            j = ii - c_sel * NP
            score = jnp.where(br, m1, m2)
            tbrow = tb_ref[0, pl.ds(j, 1), :]
            outrow = jnp.where(lane8 == 7, score, tbrow)
            outrow = jnp.where(valid, outrow, 0.0)
            out_ref[0, pl.ds(k, 1), :] = outrow
            labval = jnp.where(valid, c_sel.astype(jnp.float32), -1.0)
            lab_ref[0, pl.ds(k, 1), :] = jnp.full((1, 8), 1.0) * labval
            psl = p_ref[pl.ds(c_sel, 1)]
            p_ref[pl.ds(c_sel, 1)] = jnp.where(jiota[None] == j, NEG_INF, psl)
            return carry

        jax.lax.fori_loop(0, MAX_NUM, selbody, jnp.int32(0))

    return body


def kernel(results):
    B, N, _ = results.shape
    NP = ((N + 127) // 128) * 128
    R = NP // 128

    r = results
    x = r[..., 0]
    y = r[..., 1]
    z = r[..., 2]
    l = r[..., 3]
    w = r[..., 4]
    h = r[..., 5]
    theta_raw = r[..., 6]
    dircls = r[..., 10]
    x1 = x - l / 2
    y1 = y - w / 2
    x2 = x + l / 2
    y2 = y + w / 2
    area = (x2 - x1) * (y2 - y1)
    theta = theta_raw - jnp.floor(theta_raw / math.pi + 1) * math.pi
    theta = theta + (1.0 - dircls) * math.pi
    s = r[..., 7:10]

    def padn(a, val=0.0):
        return jnp.pad(a, ((0, 0), (0, NP - N)), constant_values=val)

    s_p = [padn(s[..., c], NEG_INF) for c in range(NCLASSES)]
    cols = jnp.stack([padn(x1), padn(y1), padn(x2), padn(y2), padn(area)] + s_p,
                     axis=1).reshape(B, 8, R, 128)
    rows = jnp.stack([padn(x1), padn(y1), padn(x2), padn(y2), padn(area),
                      jnp.zeros((B, NP), jnp.float32),
                      jnp.zeros((B, NP), jnp.float32),
                      jnp.zeros((B, NP), jnp.float32)], axis=-1)
    tbrows = jnp.stack([padn(x), padn(y), padn(z), padn(l), padn(w), padn(h),
                        padn(theta), jnp.zeros((B, NP), jnp.float32)], axis=-1)

    out8, lab8 = pl.pallas_call(
        _make_kernel(NP),
        grid=(B,),
        in_specs=[
            pl.BlockSpec((1, 8, R, 128), lambda b: (b, 0, 0, 0)),
            pl.BlockSpec((1, NP, 8), lambda b: (b, 0, 0)),
            pl.BlockSpec((1, NP, 8), lambda b: (b, 0, 0)),
        ],
        out_specs=[
            pl.BlockSpec((1, OUTPAD, 8), lambda b: (b, 0, 0)),
            pl.BlockSpec((1, OUTPAD, 8), lambda b: (b, 0, 0)),
        ],
        out_shape=[
            jax.ShapeDtypeStruct((B, OUTPAD, 8), jnp.float32),
            jax.ShapeDtypeStruct((B, OUTPAD, 8), jnp.float32),
        ],
        scratch_shapes=[
            pltpu.VMEM((R, 128), jnp.float32),
            pltpu.VMEM((R, 128), jnp.float32),
            pltpu.VMEM((NCLASSES, R, 128), jnp.float32),
            pltpu.VMEM((NCLASSES, R, 128), jnp.float32),
            pltpu.VMEM((NCLASSES, R, 128), jnp.float32),
        ],
    )(cols, rows, tbrows)

    outs = out8[:, :MAX_NUM, :]
    labels = lab8[:, :MAX_NUM, 0].astype(jnp.int32)
    return (outs, labels)
